# baseline (speedup 1.0000x reference)
import jax
import jax.numpy as jnp
from jax import lax
from jax.experimental import pallas as pl
from jax.experimental.pallas import tpu as pltpu
from jax.experimental.pallas import tpu_sc as plsc

E = 320000
D_FEAT = 128
D_EDGE = 16
D_OUT = 17

NC = 2
NS = 16
NW = NC * NS
CHUNK = 2560
NCHUNKS = E // CHUNK
MAXCH_W = (NCHUNKS + NW - 1) // NW
L = 16
NGRP = CHUNK // L


def _edge_update(vattr_flat_hbm, edge_t_hbm, out_t_hbm,
                 out_v0, out_v1, col_v0, col_v1, idx_v,
                 esem0, esem1, gsem0, gsem1, wsem0, wsem1):
    wid = lax.axis_index("s") * NC + lax.axis_index("c")
    lanes = lax.iota(jnp.int32, L)
    bufs = ((out_v0, col_v0, esem0, gsem0, wsem0),
            (out_v1, col_v1, esem1, gsem1, wsem1))

    @plsc.parallel_loop(0, NGRP, unroll=8)
    def _idx(g):
        idx_v[pl.ds(g * L, L)] = (g * L + lanes) * D_FEAT + 1

    n = (NCHUNKS - wid + NW - 1) // NW

    def _issue(i, b):
        c0 = (wid + i * NW) * CHUNK
        out_v, col_v, esem, gsem, _ = bufs[b]
        pltpu.async_copy(edge_t_hbm.at[:, pl.ds(c0, CHUNK)],
                         out_v.at[pl.ds(0, D_EDGE), :], esem)
        pltpu.async_copy(
            vattr_flat_hbm.at[pl.ds(c0 * D_FEAT, CHUNK * D_FEAT)].at[idx_v],
            col_v, gsem)

    def _wait_reads(b):
        out_v, col_v, esem, gsem, _ = bufs[b]
        pltpu.make_async_copy(edge_t_hbm.at[:, pl.ds(0, CHUNK)],
                              out_v.at[pl.ds(0, D_EDGE), :], esem).wait()
        pltpu.make_async_copy(
            vattr_flat_hbm.at[pl.ds(0, CHUNK * D_FEAT)].at[idx_v],
            col_v, gsem).wait()

    def _write(i, b):
        out_v, _, _, _, wsem = bufs[b]
        c0 = (wid + i * NW) * CHUNK
        pltpu.async_copy(out_v, out_t_hbm.at[:, pl.ds(c0, CHUNK)], wsem)

    def _wait_write(b):
        out_v, _, _, _, wsem = bufs[b]
        pltpu.make_async_copy(out_v, out_t_hbm.at[:, pl.ds(0, CHUNK)],
                              wsem).wait()

    _issue(0, 0)

    def pair_body(p, carry):
        for b in (0, 1):
            i = 2 * p + b
            nb = 1 - b

            @pl.when(i + 1 < n)
            def _():
                @pl.when(i >= 1)
                def _():
                    _wait_write(nb)
                _issue(i + 1, nb)

            @pl.when(i < n)
            def _():
                _wait_reads(b)
                out_v, col_v = bufs[b][0], bufs[b][1]

                @plsc.parallel_loop(0, NGRP, unroll=8)
                def _col(g):
                    out_v[D_EDGE, pl.ds(g * L, L)] = col_v[pl.ds(g * L, L)]

                _write(i, b)
        return carry

    lax.fori_loop(0, (MAXCH_W + 1) // 2, pair_body, 0)
    _wait_write(0)
    _wait_write(1)


def kernel(vattr_i, vattr_j, edge_attr, g, batch):
    k = pl.kernel(
        _edge_update,
        out_type=jax.ShapeDtypeStruct((D_OUT, E), jnp.float32),
        mesh=plsc.VectorSubcoreMesh(core_axis_name="c", subcore_axis_name="s"),
        scratch_types=[
            pltpu.VMEM((D_OUT, CHUNK), jnp.float32),
            pltpu.VMEM((D_OUT, CHUNK), jnp.float32),
            pltpu.VMEM((CHUNK,), jnp.float32),
            pltpu.VMEM((CHUNK,), jnp.float32),
            pltpu.VMEM((CHUNK,), jnp.int32),
            pltpu.SemaphoreType.DMA,
            pltpu.SemaphoreType.DMA,
            pltpu.SemaphoreType.DMA,
            pltpu.SemaphoreType.DMA,
            pltpu.SemaphoreType.DMA,
            pltpu.SemaphoreType.DMA,
        ],
        compiler_params=pltpu.CompilerParams(needs_layout_passes=False),
    )
    out_t = k(vattr_j.reshape(E * D_FEAT), edge_attr.T)
    return out_t.T


# speedup vs baseline: 1.0263x; 1.0263x over previous
import jax
import jax.numpy as jnp
from jax import lax
from jax.experimental import pallas as pl
from jax.experimental.pallas import tpu as pltpu
from jax.experimental.pallas import tpu_sc as plsc

E = 320000
D_FEAT = 128
D_EDGE = 16
D_OUT = 17

NC = 2
NS = 16
NW = NC * NS
CHUNK = 1280
NCHUNKS = E // CHUNK
MAXCH_W = (NCHUNKS + NW - 1) // NW
L = 16
NGRP = CHUNK // L


def _edge_update(vattr_flat_hbm, edge_t_hbm, out_t_hbm,
                 out_v0, out_v1, col_v0, col_v1, idx_v,
                 esem0, esem1, gsem0, gsem1, wsem0, wsem1):
    wid = lax.axis_index("s") * NC + lax.axis_index("c")
    lanes = lax.iota(jnp.int32, L)
    bufs = ((out_v0, col_v0, esem0, gsem0, wsem0),
            (out_v1, col_v1, esem1, gsem1, wsem1))

    @plsc.parallel_loop(0, NGRP, unroll=8)
    def _idx(g):
        idx_v[pl.ds(g * L, L)] = (g * L + lanes) * D_FEAT + 1

    n = (NCHUNKS - wid + NW - 1) // NW

    def _issue(i, b):
        c0 = (wid + i * NW) * CHUNK
        out_v, col_v, esem, gsem, _ = bufs[b]
        pltpu.async_copy(edge_t_hbm.at[:, pl.ds(c0, CHUNK)],
                         out_v.at[pl.ds(0, D_EDGE), :], esem)
        pltpu.async_copy(
            vattr_flat_hbm.at[pl.ds(c0 * D_FEAT, CHUNK * D_FEAT)].at[idx_v],
            col_v, gsem)

    def _wait_reads(b):
        out_v, col_v, esem, gsem, _ = bufs[b]
        pltpu.make_async_copy(edge_t_hbm.at[:, pl.ds(0, CHUNK)],
                              out_v.at[pl.ds(0, D_EDGE), :], esem).wait()
        pltpu.make_async_copy(
            vattr_flat_hbm.at[pl.ds(0, CHUNK * D_FEAT)].at[idx_v],
            col_v, gsem).wait()

    def _write(i, b):
        out_v, _, _, _, wsem = bufs[b]
        c0 = (wid + i * NW) * CHUNK
        pltpu.async_copy(out_v, out_t_hbm.at[:, pl.ds(c0, CHUNK)], wsem)

    def _wait_write(b):
        out_v, _, _, _, wsem = bufs[b]
        pltpu.make_async_copy(out_v, out_t_hbm.at[:, pl.ds(0, CHUNK)],
                              wsem).wait()

    _issue(0, 0)

    def pair_body(p, carry):
        for b in (0, 1):
            i = 2 * p + b
            nb = 1 - b

            @pl.when(i + 1 < n)
            def _():
                @pl.when(i >= 1)
                def _():
                    _wait_write(nb)
                _issue(i + 1, nb)

            @pl.when(i < n)
            def _():
                _wait_reads(b)
                out_v, col_v = bufs[b][0], bufs[b][1]

                @plsc.parallel_loop(0, NGRP, unroll=8)
                def _col(g):
                    out_v[D_EDGE, pl.ds(g * L, L)] = col_v[pl.ds(g * L, L)]

                _write(i, b)
        return carry

    lax.fori_loop(0, (MAXCH_W + 1) // 2, pair_body, 0)
    _wait_write(0)
    _wait_write(1)


def kernel(vattr_i, vattr_j, edge_attr, g, batch):
    k = pl.kernel(
        _edge_update,
        out_type=jax.ShapeDtypeStruct((D_OUT, E), jnp.float32),
        mesh=plsc.VectorSubcoreMesh(core_axis_name="c", subcore_axis_name="s"),
        scratch_types=[
            pltpu.VMEM((D_OUT, CHUNK), jnp.float32),
            pltpu.VMEM((D_OUT, CHUNK), jnp.float32),
            pltpu.VMEM((CHUNK,), jnp.float32),
            pltpu.VMEM((CHUNK,), jnp.float32),
            pltpu.VMEM((CHUNK,), jnp.int32),
            pltpu.SemaphoreType.DMA,
            pltpu.SemaphoreType.DMA,
            pltpu.SemaphoreType.DMA,
            pltpu.SemaphoreType.DMA,
            pltpu.SemaphoreType.DMA,
            pltpu.SemaphoreType.DMA,
        ],
        compiler_params=pltpu.CompilerParams(needs_layout_passes=False),
    )
    out_t = k(vattr_j.reshape(E * D_FEAT), edge_attr.T)
    return out_t.T
